# baseline (speedup 1.0000x reference)
import functools

import jax
import jax.numpy as jnp
from jax import lax
from jax.experimental import pallas as pl
from jax.experimental.pallas import tpu as pltpu
from jax.experimental.pallas import tpu_sc as plsc

B, S, D = 4, 2048, 1024
B_SC = 1
NC, NS = 2, 16
NW = NC * NS
S_PER_W = S // NW
CHUNK = 16
NBUF = 3
CHUNKS_PER_B = S_PER_W // CHUNK
N_CHUNKS = B_SC * CHUNKS_PER_B
CHUNK_ELEMS = CHUNK * D

TC_BS = 256


def _sc_body(x_hbm, tab_hbm, out_hbm, tab_buf, xbs, sem_t, sis, sos):
    wid = lax.axis_index("s") * NC + lax.axis_index("c")
    slab_row = wid * S_PER_W

    def xrow(c):
        b, cb = divmod(c, CHUNKS_PER_B)
        return b * S + slab_row + cb * CHUNK

    tab_d = pltpu.async_copy(
        tab_hbm.at[pl.ds(slab_row, S_PER_W)], tab_buf, sem_t)

    in_d = {}
    out_d = {}
    for c in range(NBUF - 1):
        in_d[c] = pltpu.async_copy(
            x_hbm.at[pl.ds(xrow(c), CHUNK)], xbs[c], sis[c])
    tab_d.wait()

    for c in range(N_CHUNKS):
        cq = c + NBUF - 1
        if cq < N_CHUNKS:
            q = cq % NBUF
            if cq - NBUF >= 0:
                out_d[cq - NBUF].wait()
            in_d[cq] = pltpu.async_copy(
                x_hbm.at[pl.ds(xrow(cq), CHUNK)], xbs[q], sis[q])
        p = c % NBUF
        in_d[c].wait()
        tr0 = (c % CHUNKS_PER_B) * CHUNK

        @plsc.parallel_loop(0, CHUNK_ELEMS, step=16, unroll=8)
        def _add(k, _p=p, _tr0=tr0):
            r = lax.shift_right_logical(k, 10)
            col = pl.multiple_of(lax.bitwise_and(k, D - 1), 16)
            plsc.addupdate(xbs[_p].at[r, pl.ds(col, 16)],
                           tab_buf[_tr0 + r, pl.ds(col, 16)])

        out_d[c] = pltpu.async_copy(
            xbs[p], out_hbm.at[pl.ds(xrow(c), CHUNK)], sos[p])

    for c in range(N_CHUNKS - NBUF, N_CHUNKS):
        out_d[c].wait()


def _tc_body(x_ref, tab_ref, o_ref):
    o_ref[...] = x_ref[...] + tab_ref[...]


@jax.jit
def _pe(x3, x2, tab):
    mesh = plsc.VectorSubcoreMesh(core_axis_name="c", subcore_axis_name="s")
    sc = functools.partial(
        pl.kernel,
        mesh=mesh,
        out_type=jax.ShapeDtypeStruct((B * S, D), jnp.float32),
        compiler_params=pltpu.CompilerParams(
            use_tc_tiling_on_sc=True, skip_device_barrier=True),
        scratch_types=[
            pltpu.VMEM((S_PER_W, D), jnp.float32),
            [pltpu.VMEM((CHUNK, D), jnp.float32) for _ in range(NBUF)],
            pltpu.SemaphoreType.DMA,
            [pltpu.SemaphoreType.DMA for _ in range(NBUF)],
            [pltpu.SemaphoreType.DMA for _ in range(NBUF)],
        ],
    )(_sc_body)
    sc_out = sc(x2, tab)

    tc_out = pl.pallas_call(
        _tc_body,
        grid=(B - B_SC, S // TC_BS),
        in_specs=[
            pl.BlockSpec((1, TC_BS, D), lambda b, s: (b + B_SC, s, 0)),
            pl.BlockSpec((TC_BS, D), lambda b, s: (s, 0)),
        ],
        out_specs=pl.BlockSpec((1, TC_BS, D), lambda b, s: (b, s, 0)),
        out_shape=jax.ShapeDtypeStruct((B - B_SC, S, D), jnp.float32),
    )(x3, tab)

    out = lax.dynamic_update_slice(
        sc_out.reshape(B, S, D), tc_out, (B_SC, 0, 0))
    return out


def kernel(x, pos_table):
    return _pe(x, x.reshape(B * S, D), pos_table)


# speedup vs baseline: 1.2624x; 1.2624x over previous
import functools

import jax
import jax.numpy as jnp
from jax import lax
from jax.experimental import pallas as pl
from jax.experimental.pallas import tpu as pltpu
from jax.experimental.pallas import tpu_sc as plsc

B, S, D = 4, 2048, 1024
B_SC = 3
NC, NS = 2, 16
NW = NC * NS
S_PER_W = S // NW
CHUNK = 16
NBUF = 3
CHUNKS_PER_B = S_PER_W // CHUNK
N_CHUNKS = B_SC * CHUNKS_PER_B
CHUNK_ELEMS = CHUNK * D

TC_BS = 512


def _sc_body(x_hbm, tab_hbm, out_hbm, tab_buf, xbs, sem_t, sis, sos):
    wid = lax.axis_index("s") * NC + lax.axis_index("c")
    slab_row = wid * S_PER_W

    def xrow(c):
        b, cb = divmod(c, CHUNKS_PER_B)
        return b * S + slab_row + cb * CHUNK

    tab_d = pltpu.async_copy(
        tab_hbm.at[pl.ds(slab_row, S_PER_W)], tab_buf, sem_t)

    in_d = {}
    out_d = {}
    for c in range(NBUF - 1):
        in_d[c] = pltpu.async_copy(
            x_hbm.at[pl.ds(xrow(c), CHUNK)], xbs[c], sis[c])
    tab_d.wait()

    for c in range(N_CHUNKS):
        cq = c + NBUF - 1
        if cq < N_CHUNKS:
            q = cq % NBUF
            if cq - NBUF >= 0:
                out_d[cq - NBUF].wait()
            in_d[cq] = pltpu.async_copy(
                x_hbm.at[pl.ds(xrow(cq), CHUNK)], xbs[q], sis[q])
        p = c % NBUF
        in_d[c].wait()
        tr0 = (c % CHUNKS_PER_B) * CHUNK

        @plsc.parallel_loop(0, CHUNK_ELEMS, step=16, unroll=8)
        def _add(k, _p=p, _tr0=tr0):
            r = lax.shift_right_logical(k, 10)
            col = pl.multiple_of(lax.bitwise_and(k, D - 1), 16)
            plsc.addupdate(xbs[_p].at[r, pl.ds(col, 16)],
                           tab_buf[_tr0 + r, pl.ds(col, 16)])

        out_d[c] = pltpu.async_copy(
            xbs[p], out_hbm.at[pl.ds(xrow(c), CHUNK)], sos[p])

    for c in range(N_CHUNKS - NBUF, N_CHUNKS):
        out_d[c].wait()


def _tc_body(x_ref, tab_ref, o_ref):
    o_ref[...] = x_ref[...] + tab_ref[...]


@jax.jit
def _pe(x3, x2, tab):
    mesh = plsc.VectorSubcoreMesh(core_axis_name="c", subcore_axis_name="s")
    sc = functools.partial(
        pl.kernel,
        mesh=mesh,
        out_type=jax.ShapeDtypeStruct((B * S, D), jnp.float32),
        compiler_params=pltpu.CompilerParams(
            use_tc_tiling_on_sc=True, skip_device_barrier=True),
        scratch_types=[
            pltpu.VMEM((S_PER_W, D), jnp.float32),
            [pltpu.VMEM((CHUNK, D), jnp.float32) for _ in range(NBUF)],
            pltpu.SemaphoreType.DMA,
            [pltpu.SemaphoreType.DMA for _ in range(NBUF)],
            [pltpu.SemaphoreType.DMA for _ in range(NBUF)],
        ],
    )(_sc_body)
    sc_out = sc(x2, tab)

    tc_out = pl.pallas_call(
        _tc_body,
        grid=(S // TC_BS, B - B_SC),
        in_specs=[
            pl.BlockSpec((1, TC_BS, D), lambda s, b: (b + B_SC, s, 0)),
            pl.BlockSpec((TC_BS, D), lambda s, b: (s, 0)),
        ],
        out_specs=pl.BlockSpec((1, TC_BS, D), lambda s, b: (b, s, 0)),
        out_shape=jax.ShapeDtypeStruct((B - B_SC, S, D), jnp.float32),
    )(x3, tab)

    out = lax.dynamic_update_slice(
        sc_out.reshape(B, S, D), tc_out, (B_SC, 0, 0))
    return out


def kernel(x, pos_table):
    return _pe(x, x.reshape(B * S, D), pos_table)


# speedup vs baseline: 1.2627x; 1.0003x over previous
import functools

import jax
import jax.numpy as jnp
from jax import lax
from jax.experimental import pallas as pl
from jax.experimental.pallas import tpu as pltpu
from jax.experimental.pallas import tpu_sc as plsc

B, S, D = 4, 2048, 1024
B_SC = 3
NC, NS = 2, 16
NW = NC * NS
S_PER_W = S // NW
CHUNK = 16
NBUF = 3
CHUNKS_PER_B = S_PER_W // CHUNK
N_CHUNKS = B_SC * CHUNKS_PER_B
CHUNK_ELEMS = CHUNK * D

TC_BS = 512


def _sc_body(x_hbm, tab_hbm, out_hbm, tab_buf, xbs, sem_t, sis, sos):
    wid = lax.axis_index("s") * NC + lax.axis_index("c")
    slab_row = wid * S_PER_W

    def xrow(c):
        b, cb = divmod(c, CHUNKS_PER_B)
        return b * S + slab_row + cb * CHUNK

    tab_d = pltpu.async_copy(
        tab_hbm.at[pl.ds(slab_row, S_PER_W)], tab_buf, sem_t)

    in_d = {}
    out_d = {}
    for c in range(NBUF - 1):
        in_d[c] = pltpu.async_copy(
            x_hbm.at[pl.ds(xrow(c), CHUNK)], xbs[c], sis[c])
    tab_d.wait()

    for c in range(N_CHUNKS):
        cq = c + NBUF - 1
        if cq < N_CHUNKS:
            q = cq % NBUF
            if cq - NBUF >= 0:
                out_d[cq - NBUF].wait()
            in_d[cq] = pltpu.async_copy(
                x_hbm.at[pl.ds(xrow(cq), CHUNK)], xbs[q], sis[q])
        p = c % NBUF
        in_d[c].wait()
        tr0 = (c % CHUNKS_PER_B) * CHUNK

        @plsc.parallel_loop(0, CHUNK_ELEMS, step=16, unroll=8)
        def _add(k, _p=p, _tr0=tr0):
            r = lax.shift_right_logical(k, 10)
            col = pl.multiple_of(lax.bitwise_and(k, D - 1), 16)
            plsc.addupdate(xbs[_p].at[r, pl.ds(col, 16)],
                           tab_buf[_tr0 + r, pl.ds(col, 16)])

        out_d[c] = pltpu.async_copy(
            xbs[p], out_hbm.at[pl.ds(xrow(c), CHUNK)], sos[p])

    for c in range(N_CHUNKS - NBUF, N_CHUNKS):
        out_d[c].wait()


def _tc_body(x_ref, tab_ref, o_ref):
    o_ref[...] = x_ref[...] + tab_ref[...]


@jax.jit
def _pe(x3, x2, tab):
    mesh = plsc.VectorSubcoreMesh(core_axis_name="c", subcore_axis_name="s")
    sc = functools.partial(
        pl.kernel,
        mesh=mesh,
        out_type=jax.ShapeDtypeStruct((B * S, D), jnp.float32),
        compiler_params=pltpu.CompilerParams(
            use_tc_tiling_on_sc=True, skip_device_barrier=True),
        scratch_types=[
            pltpu.VMEM((S_PER_W, D), jnp.float32),
            [pltpu.VMEM((CHUNK, D), jnp.float32) for _ in range(NBUF)],
            pltpu.SemaphoreType.DMA,
            [pltpu.SemaphoreType.DMA for _ in range(NBUF)],
            [pltpu.SemaphoreType.DMA for _ in range(NBUF)],
        ],
    )(_sc_body)

    n_tc = B - B_SC
    tc_bytes = n_tc * S * D * 4 * 2 + S * D * 4
    tc_out = pl.pallas_call(
        _tc_body,
        grid=(S // TC_BS, n_tc),
        in_specs=[
            pl.BlockSpec((1, TC_BS, D), lambda s, b: (b + B_SC, s, 0)),
            pl.BlockSpec((TC_BS, D), lambda s, b: (s, 0)),
        ],
        out_specs=pl.BlockSpec((1, TC_BS, D), lambda s, b: (b, s, 0)),
        out_shape=jax.ShapeDtypeStruct((n_tc, S, D), jnp.float32),
        cost_estimate=pl.CostEstimate(
            flops=n_tc * S * D, bytes_accessed=tc_bytes, transcendentals=0),
    )(x3, tab)

    sc_out = sc(x2, tab)

    out = lax.dynamic_update_slice(
        sc_out.reshape(B, S, D), tc_out, (B_SC, 0, 0))
    return out


def kernel(x, pos_table):
    return _pe(x, x.reshape(B * S, D), pos_table)


# speedup vs baseline: 1.2675x; 1.0038x over previous
import functools

import jax
import jax.numpy as jnp
from jax import lax
from jax.experimental import pallas as pl
from jax.experimental.pallas import tpu as pltpu
from jax.experimental.pallas import tpu_sc as plsc

B, S, D = 4, 2048, 1024
B_SC = 3
NC, NS = 2, 16
NW = NC * NS
S_PER_W = S // NW
CHUNK = 16
NBUF = 3
CHUNKS_PER_B = S_PER_W // CHUNK
N_CHUNKS = B_SC * CHUNKS_PER_B
CHUNK_ELEMS = CHUNK * D

TC_BS = 1024


def _sc_body(x_hbm, tab_hbm, out_hbm, tab_buf, xbs, sem_t, sis, sos):
    wid = lax.axis_index("s") * NC + lax.axis_index("c")
    slab_row = wid * S_PER_W

    def xrow(c):
        b, cb = divmod(c, CHUNKS_PER_B)
        return b * S + slab_row + cb * CHUNK

    tab_d = pltpu.async_copy(
        tab_hbm.at[pl.ds(slab_row, S_PER_W)], tab_buf, sem_t)

    in_d = {}
    out_d = {}
    for c in range(NBUF - 1):
        in_d[c] = pltpu.async_copy(
            x_hbm.at[pl.ds(xrow(c), CHUNK)], xbs[c], sis[c])
    tab_d.wait()

    for c in range(N_CHUNKS):
        cq = c + NBUF - 1
        if cq < N_CHUNKS:
            q = cq % NBUF
            if cq - NBUF >= 0:
                out_d[cq - NBUF].wait()
            in_d[cq] = pltpu.async_copy(
                x_hbm.at[pl.ds(xrow(cq), CHUNK)], xbs[q], sis[q])
        p = c % NBUF
        in_d[c].wait()
        tr0 = (c % CHUNKS_PER_B) * CHUNK

        @plsc.parallel_loop(0, CHUNK_ELEMS, step=16, unroll=8)
        def _add(k, _p=p, _tr0=tr0):
            r = lax.shift_right_logical(k, 10)
            col = pl.multiple_of(lax.bitwise_and(k, D - 1), 16)
            plsc.addupdate(xbs[_p].at[r, pl.ds(col, 16)],
                           tab_buf[_tr0 + r, pl.ds(col, 16)])

        out_d[c] = pltpu.async_copy(
            xbs[p], out_hbm.at[pl.ds(xrow(c), CHUNK)], sos[p])

    for c in range(N_CHUNKS - NBUF, N_CHUNKS):
        out_d[c].wait()


def _tc_body(x_ref, tab_ref, o_ref):
    o_ref[...] = x_ref[...] + tab_ref[...]


@jax.jit
def _pe(x3, x2, tab):
    mesh = plsc.VectorSubcoreMesh(core_axis_name="c", subcore_axis_name="s")
    sc = functools.partial(
        pl.kernel,
        mesh=mesh,
        out_type=jax.ShapeDtypeStruct((B * S, D), jnp.float32),
        compiler_params=pltpu.CompilerParams(
            use_tc_tiling_on_sc=True, skip_device_barrier=True),
        scratch_types=[
            pltpu.VMEM((S_PER_W, D), jnp.float32),
            [pltpu.VMEM((CHUNK, D), jnp.float32) for _ in range(NBUF)],
            pltpu.SemaphoreType.DMA,
            [pltpu.SemaphoreType.DMA for _ in range(NBUF)],
            [pltpu.SemaphoreType.DMA for _ in range(NBUF)],
        ],
    )(_sc_body)

    n_tc = B - B_SC
    tc_out = pl.pallas_call(
        _tc_body,
        grid=(n_tc, S // TC_BS),
        in_specs=[
            pl.BlockSpec((1, TC_BS, D), lambda b, s: (b + B_SC, s, 0)),
            pl.BlockSpec((TC_BS, D), lambda b, s: (s, 0)),
        ],
        out_specs=pl.BlockSpec((1, TC_BS, D), lambda b, s: (b, s, 0)),
        out_shape=jax.ShapeDtypeStruct((n_tc, S, D), jnp.float32),
    )(x3, tab)

    sc_out = sc(x2, tab)

    out = lax.dynamic_update_slice(
        sc_out.reshape(B, S, D), tc_out, (B_SC, 0, 0))
    return out


def kernel(x, pos_table):
    return _pe(x, x.reshape(B * S, D), pos_table)
